# baseline (speedup 1.0000x reference)
import functools

import jax
import jax.numpy as jnp
from jax import lax
from jax.experimental import pallas as pl
from jax.experimental.pallas import tpu as pltpu
from jax.experimental.pallas import tpu_sc as plsc

HIDDEN = 64
D0 = 64
D1 = 16
LO1 = 100000
HI1 = 900000
NC = 2
NS = 16
LANES = 16
NW = NC * NS
CH = 80
CPG = 2
GROUP = CH * CPG
NBUF = 4
ZPAD = 2048
WIDE = 512


def _sc_gather(ids, emb0, emb1):
    n = ids.shape[0]
    per_w = n // NW
    ng = per_w // GROUP
    pad = 2 * GROUP
    mesh = plsc.VectorSubcoreMesh(
        core_axis_name="c", subcore_axis_name="s",
        num_cores=NC, num_subcores=NS)

    @functools.partial(
        pl.kernel,
        out_type=(
            jax.ShapeDtypeStruct((n, D0), jnp.float32),
            jax.ShapeDtypeStruct((n, D1), jnp.float32),
        ),
        mesh=mesh,
        compiler_params=pltpu.CompilerParams(
            use_tc_tiling_on_sc=False, needs_layout_passes=False),
        scratch_types=[
            pltpu.VMEM((per_w + pad,), jnp.int32),
            pltpu.VMEM((per_w + pad,), jnp.int32),
            pltpu.VMEM((per_w,), jnp.float32),
            pltpu.VMEM((NBUF, GROUP, D0), jnp.float32),
            pltpu.VMEM((NBUF, GROUP, D1), jnp.float32),
            [pltpu.SemaphoreType.DMA] * NBUF,
            [pltpu.SemaphoreType.DMA] * NBUF,
        ],
    )
    def body(ids_hbm, emb0_hbm, emb1_hbm, rows0_hbm, rows1_hbm,
             idx0_v, idx1_v, maskv, r0, r1, gsems, ssems):
        wid = lax.axis_index("s") * NC + lax.axis_index("c")
        base = wid * per_w
        pltpu.sync_copy(ids_hbm.at[pl.ds(base, per_w)], idx0_v.at[pl.ds(0, per_w)])

        def idx_body(i, carry):
            sl = pl.ds(i * LANES, LANES)
            v = idx0_v[sl]
            m = v < LO1
            idx1_v[sl] = jnp.where(m, v & (ZPAD - 1), v - LO1)
            idx0_v[sl] = jnp.where(m, v, LO1 + (v & (ZPAD - 1)))
            maskv[sl] = jnp.where(m, 0.0, 1.0)
            return carry

        lax.fori_loop(0, per_w // LANES, idx_body, 0)

        def pad_body(i, carry):
            sl = pl.ds(per_w + i * LANES, LANES)
            v = lax.iota(jnp.int32, LANES) + i * LANES
            idx0_v[sl] = v
            idx1_v[sl] = v
            return carry

        lax.fori_loop(0, pad // LANES, pad_body, 0)

        def fire_gather(g, b):
            for k in range(CPG):
                off = g * GROUP + k * CH
                pltpu.async_copy(
                    emb0_hbm.at[idx0_v.at[pl.ds(off, CH)]],
                    r0.at[b].at[pl.ds(k * CH, CH)], gsems[b])
                pltpu.async_copy(
                    emb1_hbm.at[idx1_v.at[pl.ds(off, CH)]],
                    r1.at[b].at[pl.ds(k * CH, CH)], gsems[b])

        def wait_gather(b):
            pltpu.make_async_copy(
                rows0_hbm.at[pl.ds(0, GROUP)], r0.at[b], gsems[b]).wait()
            pltpu.make_async_copy(
                rows1_hbm.at[pl.ds(0, GROUP)], r1.at[b], gsems[b]).wait()

        def fire_scatter(g, b):
            out = pl.ds(base + g * GROUP, GROUP)
            pltpu.async_copy(r0.at[b], rows0_hbm.at[out], ssems[b])
            pltpu.async_copy(r1.at[b], rows1_hbm.at[out], ssems[b])

        def wait_scatter(b):
            pltpu.make_async_copy(
                r0.at[b], rows0_hbm.at[pl.ds(0, GROUP)], ssems[b]).wait()
            pltpu.make_async_copy(
                r1.at[b], rows1_hbm.at[pl.ds(0, GROUP)], ssems[b]).wait()

        iota = lax.iota(jnp.int32, LANES)

        def zero_bucket0(g, b):
            r1b = r1.at[b]

            def kb_body(kb, carry):
                mk = maskv[pl.ds(g * GROUP + kb * LANES, LANES)]
                rows = iota + kb * LANES
                for k in range(D1):
                    cols = jnp.full((LANES,), k, jnp.int32)
                    vals = plsc.load_gather(r1b, [rows, cols])
                    plsc.store_scatter(r1b, [rows, cols], vals * mk)
                return carry

            lax.fori_loop(0, GROUP // LANES, kb_body, 0)

        def step(g, b, first):
            bn = (b + 2) % NBUF
            if not first:
                wait_scatter(bn)
            fire_gather(g + 2, bn)
            wait_gather(b)
            zero_bucket0(g, b)
            fire_scatter(g, b)

        fire_gather(0, 0)
        fire_gather(1, 1)
        step(0, 0, True)
        step(1, 1, True)
        step(2, 2, False)
        step(3, 3, False)

        def outer(jo, carry):
            for b in range(NBUF):
                step(jo * NBUF + b, b, False)
            return carry

        lax.fori_loop(1, ng // NBUF, outer, 0)

        wait_gather(ng % NBUF)
        wait_gather((ng + 1) % NBUF)
        wait_scatter((ng - 2) % NBUF)
        wait_scatter((ng - 1) % NBUF)

    return body(ids, emb0, emb1)


def _sc_transpose_emb1(emb1t):
    v = emb1t.shape[1]
    pw = 2000
    n_panels = v // pw
    mesh = plsc.VectorSubcoreMesh(
        core_axis_name="c", subcore_axis_name="s",
        num_cores=NC, num_subcores=NS)

    @functools.partial(
        pl.kernel,
        out_type=jax.ShapeDtypeStruct((v, D1), jnp.float32),
        mesh=mesh,
        compiler_params=pltpu.CompilerParams(
            use_tc_tiling_on_sc=False, needs_layout_passes=False),
        scratch_types=[
            pltpu.VMEM((D1, pw), jnp.float32),
            pltpu.VMEM((pw, D1), jnp.float32),
        ],
    )
    def body(emb1t_hbm, out_hbm, panel_v, outp_v):
        wid = lax.axis_index("s") * NC + lax.axis_index("c")
        n_mine = (n_panels // NW) + jnp.where(wid < n_panels % NW, 1, 0)
        iota = lax.iota(jnp.int32, LANES)

        def panel_body(j, carry):
            p = wid + j * NW
            col0 = p * pw
            pltpu.sync_copy(emb1t_hbm.at[:, pl.ds(col0, pw)], panel_v)

            def blk_body(tb, c2):
                t0 = tb * LANES
                rows = iota + t0
                for k in range(D1):
                    vals = panel_v[k, pl.ds(t0, LANES)]
                    plsc.store_scatter(
                        outp_v, [rows, jnp.full((LANES,), k, jnp.int32)],
                        vals)
                return c2

            lax.fori_loop(0, pw // LANES, blk_body, 0)
            pltpu.sync_copy(outp_v, out_hbm.at[pl.ds(col0, pw)])
            return carry

        lax.fori_loop(0, n_mine, panel_body, 0)

    return body(emb1t)


def _tc_combine(rows0_w, rows1_w, factor_big):
    nw = rows0_w.shape[0]
    bt = 256

    def body(r0_ref, r1_ref, f_ref, o_ref):
        o_ref[...] = r0_ref[...] + jnp.dot(
            r1_ref[...], f_ref[...], preferred_element_type=jnp.float32)

    return pl.pallas_call(
        body,
        grid=(nw // bt,),
        in_specs=[
            pl.BlockSpec((bt, WIDE), lambda i: (i, 0)),
            pl.BlockSpec((bt, 8 * D1), lambda i: (i, 0)),
            pl.BlockSpec((8 * D1, WIDE), lambda i: (0, 0)),
        ],
        out_specs=pl.BlockSpec((bt, WIDE), lambda i: (i, 0)),
        out_shape=jax.ShapeDtypeStruct((nw, WIDE), jnp.float32),
    )(rows0_w, rows1_w, factor_big)


def kernel(input_ids, emb0, emb1, factor1):
    n = input_ids.shape[0] * input_ids.shape[1]
    ids = input_ids.reshape(-1).astype(jnp.int32)
    emb0x = jnp.concatenate(
        [emb0, jnp.zeros((ZPAD, D0), jnp.float32)], axis=0)
    emb1lin = _sc_transpose_emb1(emb1.T)
    rows0, rows1 = _sc_gather(ids, emb0x, emb1lin)
    k_ids = jnp.arange(8 * D1) // D1
    h_ids = jnp.arange(WIDE) // HIDDEN
    factor_big = jnp.where(
        (k_ids[:, None] == h_ids[None, :]),
        jnp.tile(factor1, (8, 8)), 0.0)
    out_w = _tc_combine(
        rows0.reshape(n // 8, WIDE),
        rows1.reshape(n // 8, 8 * D1),
        factor_big)
    return out_w.reshape(input_ids.shape + (HIDDEN,))


# speedup vs baseline: 2.1121x; 2.1121x over previous
import functools

import jax
import jax.numpy as jnp
from jax import lax
from jax.experimental import pallas as pl
from jax.experimental.pallas import tpu as pltpu
from jax.experimental.pallas import tpu_sc as plsc

HIDDEN = 64
D0 = 64
D1 = 16
LO1 = 100000
HI1 = 900000
NC = 2
NS = 16
LANES = 16
NW = NC * NS
GROUP = 80
NBUF = 4
ZPAD = 2048
WIDE = 512


def _sc_gather(ids, emb0, emb1w):
    n = ids.shape[0]
    per_w = n // NW
    ng = per_w // GROUP
    pad = 2 * GROUP
    mesh = plsc.VectorSubcoreMesh(
        core_axis_name="c", subcore_axis_name="s",
        num_cores=NC, num_subcores=NS)

    @functools.partial(
        pl.kernel,
        out_type=(
            jax.ShapeDtypeStruct((n, D0), jnp.float32),
            jax.ShapeDtypeStruct((n, D1), jnp.float32),
        ),
        mesh=mesh,
        compiler_params=pltpu.CompilerParams(
            use_tc_tiling_on_sc=False, needs_layout_passes=False),
        scratch_types=[
            pltpu.VMEM((per_w + pad,), jnp.int32),
            pltpu.VMEM((per_w + pad,), jnp.int32),
            pltpu.VMEM((per_w,), jnp.int32),
            pltpu.VMEM((per_w,), jnp.float32),
            pltpu.VMEM((NBUF, GROUP, D0), jnp.float32),
            pltpu.VMEM((NBUF, GROUP, 2 * D0), jnp.float32),
            pltpu.VMEM((NBUF, GROUP, D1), jnp.float32),
            [pltpu.SemaphoreType.DMA] * NBUF,
            [pltpu.SemaphoreType.DMA] * NBUF,
        ],
    )
    def body(ids_hbm, emb0_hbm, emb1w_hbm, rows0_hbm, rows1_hbm,
             idx0_v, idx1_v, subv, maskv, r0, r1l, r1c, gsems, ssems):
        wid = lax.axis_index("s") * NC + lax.axis_index("c")
        base = wid * per_w
        pltpu.sync_copy(ids_hbm.at[pl.ds(base, per_w)], idx0_v.at[pl.ds(0, per_w)])

        def idx_body(i, carry):
            sl = pl.ds(i * LANES, LANES)
            v = idx0_v[sl]
            m = v < LO1
            r1idx = v - LO1
            idx1_v[sl] = jnp.where(
                m, v & (ZPAD - 1), lax.shift_right_logical(r1idx, 3))
            subv[sl] = jnp.where(m, 0, (r1idx & 7) * D1)
            idx0_v[sl] = jnp.where(m, v, LO1 + (v & (ZPAD - 1)))
            maskv[sl] = jnp.where(m, 0.0, 1.0)
            return carry

        lax.fori_loop(0, per_w // LANES, idx_body, 0)

        def pad_body(i, carry):
            sl = pl.ds(per_w + i * LANES, LANES)
            v = lax.iota(jnp.int32, LANES) + i * LANES
            idx0_v[sl] = v
            idx1_v[sl] = v
            return carry

        lax.fori_loop(0, pad // LANES, pad_body, 0)

        def fire_gather(g, b):
            off = g * GROUP
            pltpu.async_copy(
                emb0_hbm.at[idx0_v.at[pl.ds(off, GROUP)]], r0.at[b], gsems[b])
            pltpu.async_copy(
                emb1w_hbm.at[idx1_v.at[pl.ds(off, GROUP)]], r1l.at[b],
                gsems[b])

        def wait_gather(b):
            pltpu.make_async_copy(
                emb0_hbm.at[pl.ds(0, GROUP)], r0.at[b], gsems[b]).wait()
            pltpu.make_async_copy(
                emb1w_hbm.at[pl.ds(0, GROUP)], r1l.at[b], gsems[b]).wait()

        def fire_scatter(g, b):
            out = pl.ds(base + g * GROUP, GROUP)
            pltpu.async_copy(r0.at[b], rows0_hbm.at[out], ssems[b])
            pltpu.async_copy(r1c.at[b], rows1_hbm.at[out], ssems[b])

        def wait_scatter(b):
            pltpu.make_async_copy(
                r0.at[b], rows0_hbm.at[pl.ds(0, GROUP)], ssems[b]).wait()
            pltpu.make_async_copy(
                r1c.at[b], rows1_hbm.at[pl.ds(0, GROUP)], ssems[b]).wait()

        iota = lax.iota(jnp.int32, LANES)

        def extract_rows(g, b):
            r1lb = r1l.at[b]
            r1cb = r1c.at[b]

            def kb_body(kb, carry):
                t0 = g * GROUP + kb * LANES
                mk = maskv[pl.ds(t0, LANES)]
                sub = subv[pl.ds(t0, LANES)]
                rows = iota + kb * LANES
                for k in range(D1):
                    vals = plsc.load_gather(r1lb, [rows, sub + k])
                    plsc.store_scatter(
                        r1cb, [rows, jnp.full((LANES,), k, jnp.int32)],
                        vals * mk)
                return carry

            lax.fori_loop(0, GROUP // LANES, kb_body, 0)

        def step(g, b, first):
            bn = (b + 2) % NBUF
            if not first:
                wait_scatter(bn)
            fire_gather(g + 2, bn)
            wait_gather(b)
            extract_rows(g, b)
            fire_scatter(g, b)

        fire_gather(0, 0)
        fire_gather(1, 1)
        step(0, 0, True)
        step(1, 1, True)
        step(2, 2, False)
        step(3, 3, False)

        def outer(jo, carry):
            for b in range(NBUF):
                step(jo * NBUF + b, b, False)
            return carry

        lax.fori_loop(1, ng // NBUF, outer, 0)

        wait_gather(ng % NBUF)
        wait_gather((ng + 1) % NBUF)
        wait_scatter((ng - 2) % NBUF)
        wait_scatter((ng - 1) % NBUF)

    return body(ids, emb0, emb1w)


def _tc_combine(rows0_w, rows1_w, factor_big):
    nw = rows0_w.shape[0]
    bt = 256

    def body(r0_ref, r1_ref, f_ref, o_ref):
        o_ref[...] = r0_ref[...] + jnp.dot(
            r1_ref[...], f_ref[...], preferred_element_type=jnp.float32)

    return pl.pallas_call(
        body,
        grid=(nw // bt,),
        in_specs=[
            pl.BlockSpec((bt, WIDE), lambda i: (i, 0)),
            pl.BlockSpec((bt, 8 * D1), lambda i: (i, 0)),
            pl.BlockSpec((8 * D1, WIDE), lambda i: (0, 0)),
        ],
        out_specs=pl.BlockSpec((bt, WIDE), lambda i: (i, 0)),
        out_shape=jax.ShapeDtypeStruct((nw, WIDE), jnp.float32),
    )(rows0_w, rows1_w, factor_big)


def kernel(input_ids, emb0, emb1, factor1):
    n = input_ids.shape[0] * input_ids.shape[1]
    ids = input_ids.reshape(-1).astype(jnp.int32)
    emb0x = jnp.concatenate(
        [emb0, jnp.zeros((ZPAD, D0), jnp.float32)], axis=0)
    emb1w = emb1.reshape(emb1.shape[0] // 8, 8 * D1)
    rows0, rows1 = _sc_gather(ids, emb0x, emb1w)
    k_ids = jnp.arange(8 * D1) // D1
    h_ids = jnp.arange(WIDE) // HIDDEN
    factor_big = jnp.where(
        (k_ids[:, None] == h_ids[None, :]),
        jnp.tile(factor1, (8, 8)), 0.0)
    out_w = _tc_combine(
        rows0.reshape(n // 8, WIDE),
        rows1.reshape(n // 8, 8 * D1),
        factor_big)
    return out_w.reshape(input_ids.shape + (HIDDEN,))


# speedup vs baseline: 2.2630x; 1.0714x over previous
import functools

import jax
import jax.numpy as jnp
from jax import lax
from jax.experimental import pallas as pl
from jax.experimental.pallas import tpu as pltpu
from jax.experimental.pallas import tpu_sc as plsc

HIDDEN = 64
D0 = 64
D1 = 16
LO1 = 100000
HI1 = 900000
NC = 2
NS = 16
LANES = 16
NW = NC * NS
CH = 80
CPG = 2
GROUP = CH * CPG
NBUF = 4
ZPAD = 2048
WIDE = 512


def _sc_gather(ids, emb0, emb1):
    n = ids.shape[0]
    per_w = n // NW
    ng = per_w // GROUP
    pad = 2 * GROUP
    mesh = plsc.VectorSubcoreMesh(
        core_axis_name="c", subcore_axis_name="s",
        num_cores=NC, num_subcores=NS)

    @functools.partial(
        pl.kernel,
        out_type=(
            jax.ShapeDtypeStruct((n, D0), jnp.float32),
            jax.ShapeDtypeStruct((n, D1), jnp.float32),
        ),
        mesh=mesh,
        compiler_params=pltpu.CompilerParams(
            use_tc_tiling_on_sc=False, needs_layout_passes=False),
        scratch_types=[
            pltpu.VMEM((per_w + pad,), jnp.int32),
            pltpu.VMEM((per_w + pad,), jnp.int32),
            pltpu.VMEM((per_w,), jnp.float32),
            pltpu.VMEM((NBUF, GROUP, D0), jnp.float32),
            pltpu.VMEM((NBUF, GROUP, D1), jnp.float32),
            [pltpu.SemaphoreType.DMA] * NBUF,
            [pltpu.SemaphoreType.DMA] * NBUF,
        ],
    )
    def body(ids_hbm, emb0_hbm, emb1_hbm, rows0_hbm, rows1_hbm,
             idx0_v, idx1_v, maskv, r0, r1, gsems, ssems):
        wid = lax.axis_index("s") * NC + lax.axis_index("c")
        base = wid * per_w
        pltpu.sync_copy(ids_hbm.at[pl.ds(base, per_w)], idx0_v.at[pl.ds(0, per_w)])

        def idx_body(i, carry):
            sl = pl.ds(i * LANES, LANES)
            v = idx0_v[sl]
            m = v < LO1
            idx1_v[sl] = jnp.where(m, v & (ZPAD - 1), v - LO1)
            idx0_v[sl] = jnp.where(m, v, LO1 + (v & (ZPAD - 1)))
            maskv[sl] = jnp.where(m, 0.0, 1.0)
            return carry

        lax.fori_loop(0, per_w // LANES, idx_body, 0)

        def pad_body(i, carry):
            sl = pl.ds(per_w + i * LANES, LANES)
            v = lax.iota(jnp.int32, LANES) + i * LANES
            idx0_v[sl] = v
            idx1_v[sl] = v
            return carry

        lax.fori_loop(0, pad // LANES, pad_body, 0)

        def fire_gather(g, b):
            for k in range(CPG):
                off = g * GROUP + k * CH
                pltpu.async_copy(
                    emb0_hbm.at[idx0_v.at[pl.ds(off, CH)]],
                    r0.at[b].at[pl.ds(k * CH, CH)], gsems[b])
                pltpu.async_copy(
                    emb1_hbm.at[idx1_v.at[pl.ds(off, CH)]],
                    r1.at[b].at[pl.ds(k * CH, CH)], gsems[b])

        def wait_gather(b):
            pltpu.make_async_copy(
                rows0_hbm.at[pl.ds(0, GROUP)], r0.at[b], gsems[b]).wait()
            pltpu.make_async_copy(
                rows1_hbm.at[pl.ds(0, GROUP)], r1.at[b], gsems[b]).wait()

        def fire_scatter(g, b):
            out = pl.ds(base + g * GROUP, GROUP)
            pltpu.async_copy(r0.at[b], rows0_hbm.at[out], ssems[b])
            pltpu.async_copy(r1.at[b], rows1_hbm.at[out], ssems[b])

        def wait_scatter(b):
            pltpu.make_async_copy(
                r0.at[b], rows0_hbm.at[pl.ds(0, GROUP)], ssems[b]).wait()
            pltpu.make_async_copy(
                r1.at[b], rows1_hbm.at[pl.ds(0, GROUP)], ssems[b]).wait()

        iota = lax.iota(jnp.int32, LANES)

        def zero_bucket0(g, b):
            r1b = r1.at[b]

            def kb_body(kb, carry):
                mk = maskv[pl.ds(g * GROUP + kb * LANES, LANES)]
                rows = iota + kb * LANES
                for k in range(D1):
                    cols = jnp.full((LANES,), k, jnp.int32)
                    vals = plsc.load_gather(r1b, [rows, cols])
                    plsc.store_scatter(r1b, [rows, cols], vals * mk)
                return carry

            lax.fori_loop(0, GROUP // LANES, kb_body, 0)

        def step(g, b, first):
            bn = (b + 2) % NBUF
            if not first:
                wait_scatter(bn)
            fire_gather(g + 2, bn)
            wait_gather(b)
            zero_bucket0(g, b)
            fire_scatter(g, b)

        fire_gather(0, 0)
        fire_gather(1, 1)
        step(0, 0, True)
        step(1, 1, True)
        step(2, 2, False)
        step(3, 3, False)

        def outer(jo, carry):
            for b in range(NBUF):
                step(jo * NBUF + b, b, False)
            return carry

        lax.fori_loop(1, ng // NBUF, outer, 0)

        wait_gather(ng % NBUF)
        wait_gather((ng + 1) % NBUF)
        wait_scatter((ng - 2) % NBUF)
        wait_scatter((ng - 1) % NBUF)

    return body(ids, emb0, emb1)


def _sc_transpose_emb1(emb1t):
    v = emb1t.shape[1]
    pw = 2000
    n_panels = v // pw
    mesh = plsc.VectorSubcoreMesh(
        core_axis_name="c", subcore_axis_name="s",
        num_cores=NC, num_subcores=NS)

    @functools.partial(
        pl.kernel,
        out_type=jax.ShapeDtypeStruct((v, D1), jnp.float32),
        mesh=mesh,
        compiler_params=pltpu.CompilerParams(
            use_tc_tiling_on_sc=False, needs_layout_passes=False),
        scratch_types=[
            pltpu.VMEM((D1, pw), jnp.float32),
            pltpu.VMEM((pw, D1), jnp.float32),
        ],
    )
    def body(emb1t_hbm, out_hbm, panel_v, outp_v):
        wid = lax.axis_index("s") * NC + lax.axis_index("c")
        n_mine = (n_panels // NW) + jnp.where(wid < n_panels % NW, 1, 0)
        iota = lax.iota(jnp.int32, LANES)

        def panel_body(j, carry):
            p = wid + j * NW
            col0 = p * pw
            pltpu.sync_copy(emb1t_hbm.at[:, pl.ds(col0, pw)], panel_v)

            def blk_body(tb, c2):
                t0 = tb * LANES
                rows = iota + t0
                for k in range(D1):
                    vals = panel_v[k, pl.ds(t0, LANES)]
                    plsc.store_scatter(
                        outp_v, [rows, jnp.full((LANES,), k, jnp.int32)],
                        vals)
                return c2

            lax.fori_loop(0, pw // LANES, blk_body, 0)
            pltpu.sync_copy(outp_v, out_hbm.at[pl.ds(col0, pw)])
            return carry

        lax.fori_loop(0, n_mine, panel_body, 0)

    return body(emb1t)


def _tc_combine(rows0_w, rows1_w, factor_big):
    nw = rows0_w.shape[0]
    bt = 512

    def body(r0_ref, r1_ref, f_ref, o_ref):
        o_ref[...] = r0_ref[...] + jnp.dot(
            r1_ref[...], f_ref[...], preferred_element_type=jnp.float32)

    return pl.pallas_call(
        body,
        grid=(nw // bt,),
        in_specs=[
            pl.BlockSpec((bt, WIDE), lambda i: (i, 0)),
            pl.BlockSpec((bt, 8 * D1), lambda i: (i, 0)),
            pl.BlockSpec((8 * D1, WIDE), lambda i: (0, 0)),
        ],
        out_specs=pl.BlockSpec((bt, WIDE), lambda i: (i, 0)),
        out_shape=jax.ShapeDtypeStruct((nw, WIDE), jnp.float32),
    )(rows0_w, rows1_w, factor_big)


def kernel(input_ids, emb0, emb1, factor1):
    n = input_ids.shape[0] * input_ids.shape[1]
    ids = input_ids.reshape(-1).astype(jnp.int32)
    emb0x = jnp.pad(emb0, ((0, ZPAD), (0, 0)))
    rows0, rows1 = _sc_gather(ids, emb0x, emb1)
    k_ids = jnp.arange(8 * D1) // D1
    h_ids = jnp.arange(WIDE) // HIDDEN
    factor_big = jnp.where(
        (k_ids[:, None] == h_ids[None, :]),
        jnp.tile(factor1, (8, 8)), 0.0)
    out_w = _tc_combine(
        rows0.reshape(n // 8, WIDE),
        rows1.reshape(n // 8, 8 * D1),
        factor_big)
    return out_w.reshape(input_ids.shape + (HIDDEN,))
